# baseline (speedup 1.0000x reference)
import jax
import jax.numpy as jnp
from jax.experimental import pallas as pl
from jax.experimental.pallas import tpu as pltpu

_ROWS = 2048
_VOCAB = 100000
_R = 8
_D = 8
_NSTEP = _ROWS // _R
_W = 512
_NCHUNK = _VOCAB // _W
_TAIL = _VOCAB - _NCHUNK * _W
_BIG = 2**30


def _reduce(x_ref):
    m = x_ref[:, 0:_W]
    bi = jnp.zeros((_R, _W), jnp.int32)
    for k in range(1, _NCHUNK):
        xk = x_ref[:, _W * k:_W * (k + 1)]
        gt = xk > m
        m = jnp.maximum(m, xk)
        bi = jnp.where(gt, jnp.int32(k), bi)
    xt = x_ref[:, _NCHUNK * _W:_VOCAB]

    maxx = jnp.maximum(jnp.max(m, axis=-1), jnp.max(xt, axis=-1))
    lane = jax.lax.broadcasted_iota(jnp.int32, (_R, _W), 1)
    cand = jnp.where(m == maxx[:, None], bi * _W + lane, _BIG)
    lane_t = jax.lax.broadcasted_iota(jnp.int32, (_R, _TAIL), 1)
    cand_t = jnp.where(xt == maxx[:, None], _NCHUNK * _W + lane_t, _BIG)
    ids = jnp.minimum(jnp.min(cand, axis=-1), jnp.min(cand_t, axis=-1))
    return maxx, ids


def _slot_copy(hbm_ref, buf, sems, step, slot):
    return pltpu.make_async_copy(
        hbm_ref.at[0, pl.ds(step * _R, _R), :], buf.at[slot], sems.at[slot])


def _fwd_kernel(loss_ref, hbm_ref, loss_out_ref, max_ref, ids_ref, buf, sems):
    i = pl.program_id(0)

    @pl.when(i == 0)
    def _warmup():
        for d in range(_D):
            _slot_copy(hbm_ref, buf, sems, d, d).start()

    slot = jax.lax.rem(i, _D)
    _slot_copy(hbm_ref, buf, sems, i, slot).wait()

    maxx, ids = _reduce(buf.at[slot])
    max_ref[0, :, 0] = maxx
    ids_ref[0, :, 0] = ids
    loss_out_ref[0, 0] = loss_ref[0, 0]

    @pl.when(i + _D < _NSTEP)
    def _prefetch():
        _slot_copy(hbm_ref, buf, sems, i + _D, slot).start()


def kernel(loss, logits):
    loss2d = loss.reshape(1, 1)
    loss_out, _, _ = pl.pallas_call(
        _fwd_kernel,
        grid=(_NSTEP,),
        in_specs=[
            pl.BlockSpec(memory_space=pltpu.SMEM),
            pl.BlockSpec(memory_space=pltpu.HBM),
        ],
        out_specs=[
            pl.BlockSpec(memory_space=pltpu.SMEM),
            pl.BlockSpec((1, _R, 1), lambda i: (0, i, 0)),
            pl.BlockSpec((1, _R, 1), lambda i: (0, i, 0)),
        ],
        out_shape=[
            jax.ShapeDtypeStruct((1, 1), jnp.float32),
            jax.ShapeDtypeStruct((1, _ROWS, 1), jnp.float32),
            jax.ShapeDtypeStruct((1, _ROWS, 1), jnp.int32),
        ],
        scratch_shapes=[
            pltpu.VMEM((_D, _R, _VOCAB), jnp.float32),
            pltpu.SemaphoreType.DMA((_D,)),
        ],
        compiler_params=pltpu.CompilerParams(
            dimension_semantics=("arbitrary",),
        ),
    )(loss2d, logits)
    return loss_out.reshape(())
